# baseline (speedup 1.0000x reference)
import jax
import jax.numpy as jnp
from jax.experimental import pallas as pl
from jax.experimental.pallas import tpu as pltpu


def _make_body(streams):
    def body(*refs):
        x_ref = refs[0]
        a_refs = refs[1:1 + streams]
        w_ref = refs[1 + streams]
        out_ref = refs[2 + streams]
        h_ref = refs[3 + streams]

        @pl.when(pl.program_id(0) == 0)
        def _():
            h_ref[...] = jnp.dot(
                x_ref[...], w_ref[...], preferred_element_type=jnp.float32
            )

        bm = a_refs[0].shape[0]
        for s in range(streams):
            out_ref[s * bm:(s + 1) * bm, :] = jnp.dot(
                a_refs[s][...], h_ref[...], preferred_element_type=jnp.float32
            )

    return body


def _pick_bm(m, n, streams):
    best = 0
    for cand in range(8, min(m, 2048) + 1, 8):
        if m % (streams * cand) == 0 and \
                cand * n * 4 * 2 * streams <= 40 * 1024 * 1024:
            best = cand
    return best


def _run(x, A_loc, W, streams, bm):
    m = A_loc.shape[0]
    N, d_in = x.shape
    d_out = W.shape[1]
    h_scratch = pltpu.VMEM((N, d_out), jnp.float32)

    def a_spec(s):
        return pl.BlockSpec((bm, N), lambda i, s=s: (streams * i + s, 0))

    return pl.pallas_call(
        _make_body(streams),
        grid=(m // (streams * bm),),
        in_specs=[pl.BlockSpec((N, d_in), lambda i: (0, 0))]
        + [a_spec(s) for s in range(streams)]
        + [pl.BlockSpec((d_in, d_out), lambda i: (0, 0))],
        out_specs=pl.BlockSpec((streams * bm, d_out), lambda i: (i, 0)),
        out_shape=jax.ShapeDtypeStruct((m, d_out), jnp.float32),
        scratch_shapes=[h_scratch],
    )(x, *([A_loc] * streams), W)


def _local(x, A_loc, W):
    m = A_loc.shape[0]
    n = A_loc.shape[1]
    for streams in (5, 2, 1):
        bm = _pick_bm(m, n, streams)
        if bm >= 8:
            return _run(x, A_loc, W, streams, bm)
    return _run(x, A_loc, W, 1, m)


def kernel(x, A, W):
    return _local(x, A, W)


# speedup vs baseline: 1.0138x; 1.0138x over previous
import jax
import jax.numpy as jnp
from jax.experimental import pallas as pl
from jax.experimental.pallas import tpu as pltpu


def _make_body(streams):
    def body(*refs):
        x_ref = refs[0]
        a_refs = refs[1:1 + streams]
        w_ref = refs[1 + streams]
        out_ref = refs[2 + streams]
        h_ref = refs[3 + streams]

        @pl.when(pl.program_id(0) == 0)
        def _():
            h_ref[...] = jnp.dot(
                x_ref[...], w_ref[...], preferred_element_type=jnp.float32
            )

        bm = a_refs[0].shape[0]
        for s in range(streams):
            out_ref[s * bm:(s + 1) * bm, :] = jnp.dot(
                a_refs[s][...], h_ref[...], preferred_element_type=jnp.float32
            )

    return body


def _pick_bm(m, n, streams):
    best = 0
    for cand in range(8, min(m, 2048) + 1, 8):
        if m % (streams * cand) == 0 and \
                cand * n * 4 * 2 * streams <= 40 * 1024 * 1024:
            best = cand
    return best


def _run(x, A_loc, W, streams, bm):
    m = A_loc.shape[0]
    N, d_in = x.shape
    d_out = W.shape[1]
    h_scratch = pltpu.VMEM((N, d_out), jnp.float32)

    def a_spec(s):
        return pl.BlockSpec((bm, N), lambda i, s=s: (streams * i + s, 0))

    return pl.pallas_call(
        _make_body(streams),
        grid=(pl.cdiv(m, streams * bm),),
        in_specs=[pl.BlockSpec((N, d_in), lambda i: (0, 0))]
        + [a_spec(s) for s in range(streams)]
        + [pl.BlockSpec((d_in, d_out), lambda i: (0, 0))],
        out_specs=pl.BlockSpec((streams * bm, d_out), lambda i: (i, 0)),
        out_shape=jax.ShapeDtypeStruct((m, d_out), jnp.float32),
        scratch_shapes=[h_scratch],
    )(x, *([A_loc] * streams), W)


def _local(x, A_loc, W):
    return _run(x, A_loc, W, 2, 240)


def kernel(x, A, W):
    return _local(x, A, W)


# speedup vs baseline: 1.0231x; 1.0092x over previous
import jax
import jax.numpy as jnp
from jax.experimental import pallas as pl
from jax.experimental.pallas import tpu as pltpu


def _make_body(streams):
    def body(*refs):
        x_ref = refs[0]
        a_refs = refs[1:1 + streams]
        w_ref = refs[1 + streams]
        out_ref = refs[2 + streams]
        h_ref = refs[3 + streams]

        @pl.when(pl.program_id(0) == 0)
        def _():
            h_ref[...] = jnp.dot(
                x_ref[...], w_ref[...], preferred_element_type=jnp.float32
            )

        bm = a_refs[0].shape[0]
        for s in range(streams):
            out_ref[s * bm:(s + 1) * bm, :] = jnp.dot(
                a_refs[s][...], h_ref[...], preferred_element_type=jnp.float32
            )

    return body


def _pick_bm(m, n, streams):
    best = 0
    for cand in range(8, min(m, 2048) + 1, 8):
        if m % (streams * cand) == 0 and \
                cand * n * 4 * 2 * streams <= 40 * 1024 * 1024:
            best = cand
    return best


def _run(x, A_loc, W, streams, bm):
    m = A_loc.shape[0]
    N, d_in = x.shape
    d_out = W.shape[1]
    h_scratch = pltpu.VMEM((N, d_out), jnp.float32)

    def a_spec(s):
        return pl.BlockSpec((bm, N), lambda i, s=s: (streams * i + s, 0))

    return pl.pallas_call(
        _make_body(streams),
        grid=(pl.cdiv(m, streams * bm),),
        in_specs=[pl.BlockSpec((N, d_in), lambda i: (0, 0))]
        + [a_spec(s) for s in range(streams)]
        + [pl.BlockSpec((d_in, d_out), lambda i: (0, 0))],
        out_specs=pl.BlockSpec((streams * bm, d_out), lambda i: (i, 0)),
        out_shape=jax.ShapeDtypeStruct((m, d_out), jnp.float32),
        scratch_shapes=[h_scratch],
    )(x, *([A_loc] * streams), W)


def _local(x, A_loc, W):
    return _run(x, A_loc, W, 1, 400)


def kernel(x, A, W):
    return _local(x, A, W)
